# baseline (speedup 1.0000x reference)
import functools

import jax
import jax.numpy as jnp
from jax import lax
from jax.experimental import pallas as pl
from jax.experimental.pallas import tpu as pltpu
from jax.experimental.pallas import tpu_sc as plsc


def _segsum_sc(flat, B, P, D):
    info = plsc.get_sparse_core_info()
    NC, NS, L = info.num_cores, info.num_subcores, info.num_lanes
    NW = NC * NS
    NV = D // L
    BPW = B // NW
    RPC = 128
    SPC = RPC // P
    NCHUNK = (BPW * P) // RPC
    NBUF = 6
    AHEAD = NBUF - 1
    mesh = plsc.VectorSubcoreMesh(core_axis_name="c", subcore_axis_name="s")

    @functools.partial(
        pl.kernel,
        out_type=jax.ShapeDtypeStruct((B, D), jnp.float32),
        mesh=mesh,
        scratch_types=[
            [pltpu.VMEM((RPC, D), jnp.float32)] * NBUF,
            pltpu.VMEM((BPW, D), jnp.float32),
            [pltpu.SemaphoreType.DMA] * NBUF,
        ],
    )
    def seg(flat_hbm, out_hbm, bufs, res, hsems):
        sid = lax.axis_index("s")
        wid = lax.axis_index("c") * NS + sid
        in_base = wid * (BPW * P)

        def hbm_start(g):
            return pltpu.async_copy(
                flat_hbm.at[pl.ds(in_base + g * RPC, RPC)],
                bufs[g % NBUF], hsems[g % NBUF])

        def reduce_chunk(buf, g):
            def seg_body(t, _):
                base = t * P
                acc = [buf[base, pl.ds(j * L, L)] for j in range(NV)]
                def row_body(r, acc):
                    return tuple(
                        acc[j] + buf[base + r, pl.ds(j * L, L)]
                        for j in range(NV)
                    )
                acc = lax.fori_loop(1, P, row_body, tuple(acc))
                for j in range(NV):
                    res[g * SPC + t, pl.ds(j * L, L)] = acc[j]
                return _
            lax.fori_loop(0, SPC, seg_body, 0)

        hbm_d = [hbm_start(i) for i in range(min(AHEAD, NCHUNK))]
        for gg in range(0, NCHUNK, NBUF):
            for b in range(NBUF):
                g = gg + b
                if g >= NCHUNK:
                    break
                hbm_d.pop(0).wait()
                if g + AHEAD < NCHUNK:
                    hbm_d.append(hbm_start(g + AHEAD))
                reduce_chunk(bufs[g % NBUF], g)

        pltpu.sync_copy(res, out_hbm.at[pl.ds(wid * BPW, BPW)])

    return seg(flat)


def _mlp_body(s_ref, w_ref, b_ref, o_ref):
    y = jax.lax.dot_general(
        s_ref[...], w_ref[...], (((1,), (1,)), ((), ())),
        preferred_element_type=jnp.float32,
        precision=jax.lax.Precision.HIGHEST,
    )
    o_ref[...] = y + b_ref[...]


def kernel(pert_batch, W, b):
    B, P, D = pert_batch.shape
    OUT = W.shape[0]
    flat = pert_batch.reshape(B * P, D)
    s = _segsum_sc(flat, B, P, D)
    bias = (P * b).reshape(1, OUT)
    return pl.pallas_call(
        _mlp_body,
        in_specs=[
            pl.BlockSpec((B, D), lambda: (0, 0)),
            pl.BlockSpec((OUT, D), lambda: (0, 0)),
            pl.BlockSpec((1, OUT), lambda: (0, 0)),
        ],
        out_specs=pl.BlockSpec((B, OUT), lambda: (0, 0)),
        out_shape=jax.ShapeDtypeStruct((B, OUT), jnp.float32),
    )(s, W, bias)


# speedup vs baseline: 1.0410x; 1.0410x over previous
import functools

import jax
import jax.numpy as jnp
from jax import lax
from jax.experimental import pallas as pl
from jax.experimental.pallas import tpu as pltpu
from jax.experimental.pallas import tpu_sc as plsc


def _segsum_sc(flat, B, P, D):
    info = plsc.get_sparse_core_info()
    NC, NS, L = info.num_cores, info.num_subcores, info.num_lanes
    NW = NC * NS
    NV = D // L
    BPW = B // NW
    RPC = 256
    SPC = RPC // P
    NCHUNK = (BPW * P) // RPC
    NBUF = 3
    mesh = plsc.VectorSubcoreMesh(core_axis_name="c", subcore_axis_name="s")

    @functools.partial(
        pl.kernel,
        out_type=jax.ShapeDtypeStruct((B, D), jnp.float32),
        mesh=mesh,
        scratch_types=[
            [pltpu.VMEM((RPC, D), jnp.float32)] * NBUF,
            pltpu.VMEM((BPW, D), jnp.float32),
            [pltpu.SemaphoreType.DMA] * NBUF,
        ],
    )
    def seg(flat_hbm, out_hbm, bufs, res, hsems):
        sid = lax.axis_index("s")
        wid = lax.axis_index("c") * NS + sid
        in_base = wid * (BPW * P)

        def hbm_start(g):
            return pltpu.async_copy(
                flat_hbm.at[pl.ds(in_base + g * RPC, RPC)],
                bufs[g % NBUF], hsems[g % NBUF])

        def reduce_chunk(buf, g):
            def seg_body(t, _):
                base = t * P
                acc = [buf[base, pl.ds(j * L, L)] for j in range(NV)]
                def row_body(r, acc):
                    return tuple(
                        acc[j] + buf[base + r, pl.ds(j * L, L)]
                        for j in range(NV)
                    )
                acc = lax.fori_loop(1, P, row_body, tuple(acc))
                for j in range(NV):
                    res[g * SPC + t, pl.ds(j * L, L)] = acc[j]
                return _
            lax.fori_loop(0, SPC, seg_body, 0)

        hbm_d = [hbm_start(0), hbm_start(1)]
        for gg in range(0, NCHUNK, NBUF):
            for b in range(NBUF):
                g = gg + b
                if g >= NCHUNK:
                    break
                hbm_d.pop(0).wait()
                if g + 2 < NCHUNK:
                    hbm_d.append(hbm_start(g + 2))
                reduce_chunk(bufs[g % NBUF], g)

        pltpu.sync_copy(res, out_hbm.at[pl.ds(wid * BPW, BPW)])

    return seg(flat)


def _mlp_body(s_ref, w_ref, b_ref, o_ref):
    y = jax.lax.dot_general(
        s_ref[...], w_ref[...], (((1,), (1,)), ((), ())),
        preferred_element_type=jnp.float32,
        precision=jax.lax.Precision.HIGHEST,
    )
    o_ref[...] = y + b_ref[...]


def kernel(pert_batch, W, b):
    B, P, D = pert_batch.shape
    OUT = W.shape[0]
    flat = pert_batch.reshape(B * P, D)
    s = _segsum_sc(flat, B, P, D)
    bias = (P * b).reshape(1, OUT)
    return pl.pallas_call(
        _mlp_body,
        in_specs=[
            pl.BlockSpec((B, D), lambda: (0, 0)),
            pl.BlockSpec((OUT, D), lambda: (0, 0)),
            pl.BlockSpec((1, OUT), lambda: (0, 0)),
        ],
        out_specs=pl.BlockSpec((B, OUT), lambda: (0, 0)),
        out_shape=jax.ShapeDtypeStruct((B, OUT), jnp.float32),
    )(s, W, bias)
